# baseline (speedup 1.0000x reference)
import functools

import jax
import jax.numpy as jnp
from jax import lax
from jax.experimental import pallas as pl
from jax.experimental.pallas import tpu as pltpu
from jax.experimental.pallas import tpu_sc as plsc

N = 128 * 128 * 128
NC = 2
NS = 16
NW = NC * NS
PER_W = N // NW
CHUNK = 16384
NCHUNK = PER_W // CHUNK
LANES = 16
UNROLL = 4

_mesh = plsc.VectorSubcoreMesh(core_axis_name="c", subcore_axis_name="s")


def _loss_partials_body(pred_hbm, targ_hbm, out_hbm, pred_v, gt_v, m_v, acc_v, cnt_v):
    wid = lax.axis_index("s") * NC + lax.axis_index("c")
    base = wid * PER_W

    acc = jnp.zeros((LANES,), jnp.float32)
    cnt = jnp.zeros((LANES,), jnp.float32)

    for j in range(NCHUNK):
        off = base + j * CHUNK
        pltpu.sync_copy(pred_hbm.at[pl.ds(off, CHUNK)], pred_v)
        pltpu.sync_copy(targ_hbm.at[pl.ds(off, CHUNK)], gt_v)
        pltpu.sync_copy(targ_hbm.at[pl.ds(N + off, CHUNK)], m_v)

        def body(i, carry):
            a, c = carry
            for u in range(UNROLL):
                s = i * (LANES * UNROLL) + u * LANES
                p = pred_v[pl.ds(s, LANES)]
                g = gt_v[pl.ds(s, LANES)]
                m = m_v[pl.ds(s, LANES)]
                sel = m > 0
                d = jnp.where(sel, jnp.abs(p - g), 0.0)
                one = jnp.where(sel, 1.0, 0.0)
                a = a + d
                c = c + one
            return a, c

        acc, cnt = lax.fori_loop(0, CHUNK // (LANES * UNROLL), body, (acc, cnt))

    acc_v[...] = acc
    cnt_v[...] = cnt
    pltpu.sync_copy(acc_v, out_hbm.at[0, wid])
    pltpu.sync_copy(cnt_v, out_hbm.at[1, wid])


_loss_partials = functools.partial(
    pl.kernel,
    out_type=jax.ShapeDtypeStruct((2, NW, LANES), jnp.float32),
    mesh=_mesh,
    scratch_types=[
        pltpu.VMEM((CHUNK,), jnp.float32),
        pltpu.VMEM((CHUNK,), jnp.float32),
        pltpu.VMEM((CHUNK,), jnp.float32),
        pltpu.VMEM((LANES,), jnp.float32),
        pltpu.VMEM((LANES,), jnp.float32),
    ],
)(_loss_partials_body)


@jax.jit
def kernel(predictions, targets):
    pred_flat = predictions.reshape(-1)
    targ_flat = targets.reshape(-1)
    partials = _loss_partials(pred_flat, targ_flat)
    total = jnp.sum(partials[0])
    count = jnp.sum(partials[1])
    return total / jnp.maximum(count, 1.0)


# speedup vs baseline: 1.2842x; 1.2842x over previous
import functools

import jax
import jax.numpy as jnp
from jax import lax
from jax.experimental import pallas as pl
from jax.experimental.pallas import tpu as pltpu
from jax.experimental.pallas import tpu_sc as plsc

N = 128 * 128 * 128
NC = 2
NS = 16
NW = NC * NS
PER_W = N // NW
CHUNK = 16384
NCHUNK = PER_W // CHUNK
LANES = 16
UNROLL = 4

_mesh = plsc.VectorSubcoreMesh(core_axis_name="c", subcore_axis_name="s")


def _loss_partials_body(pred_hbm, targ_hbm, out_hbm,
                        p0, g0, m0, p1, g1, m1, acc_v, sem0, sem1):
    wid = lax.axis_index("s") * NC + lax.axis_index("c")
    base = wid * PER_W
    bufs = ((p0, g0, m0), (p1, g1, m1))
    sems = (sem0, sem1)

    def issue(j, slot):
        off = base + j * CHUNK
        pv, gv, mv = bufs[slot]
        return (
            pltpu.async_copy(pred_hbm.at[pl.ds(off, CHUNK)], pv, sems[slot]),
            pltpu.async_copy(targ_hbm.at[pl.ds(off, CHUNK)], gv, sems[slot]),
            pltpu.async_copy(targ_hbm.at[pl.ds(N + off, CHUNK)], mv, sems[slot]),
        )

    accs = [jnp.zeros((LANES,), jnp.float32) for _ in range(UNROLL)]
    cnts = [jnp.zeros((LANES,), jnp.float32) for _ in range(UNROLL)]

    pending = [None, None]
    pending[0] = issue(0, 0)
    for j in range(NCHUNK):
        slot = j & 1
        if j + 1 < NCHUNK:
            pending[(j + 1) & 1] = issue(j + 1, (j + 1) & 1)
        for d in pending[slot]:
            d.wait()
        pv, gv, mv = bufs[slot]

        def body(i, carry):
            a = list(carry[:UNROLL])
            c = list(carry[UNROLL:])
            for u in range(UNROLL):
                s = i * (LANES * UNROLL) + u * LANES
                p = pv[pl.ds(s, LANES)]
                g = gv[pl.ds(s, LANES)]
                m = mv[pl.ds(s, LANES)]
                sel = m > 0
                a[u] = a[u] + jnp.where(sel, jnp.abs(p - g), 0.0)
                c[u] = c[u] + jnp.where(sel, 1.0, 0.0)
            return tuple(a) + tuple(c)

        out = lax.fori_loop(0, CHUNK // (LANES * UNROLL), body,
                            tuple(accs) + tuple(cnts))
        accs = list(out[:UNROLL])
        cnts = list(out[UNROLL:])

    acc = accs[0] + accs[1] + accs[2] + accs[3]
    cnt = cnts[0] + cnts[1] + cnts[2] + cnts[3]
    acc_v[pl.ds(0, LANES)] = acc
    acc_v[pl.ds(LANES, LANES)] = cnt
    pltpu.sync_copy(acc_v, out_hbm.at[wid])


_loss_partials = functools.partial(
    pl.kernel,
    out_type=jax.ShapeDtypeStruct((NW, 2 * LANES), jnp.float32),
    mesh=_mesh,
    scratch_types=[
        pltpu.VMEM((CHUNK,), jnp.float32),
        pltpu.VMEM((CHUNK,), jnp.float32),
        pltpu.VMEM((CHUNK,), jnp.float32),
        pltpu.VMEM((CHUNK,), jnp.float32),
        pltpu.VMEM((CHUNK,), jnp.float32),
        pltpu.VMEM((CHUNK,), jnp.float32),
        pltpu.VMEM((2 * LANES,), jnp.float32),
        pltpu.SemaphoreType.DMA,
        pltpu.SemaphoreType.DMA,
    ],
)(_loss_partials_body)


@jax.jit
def kernel(predictions, targets):
    pred_flat = predictions.reshape(-1)
    targ_flat = targets.reshape(-1)
    partials = _loss_partials(pred_flat, targ_flat)
    total = jnp.sum(partials[:, :LANES])
    count = jnp.sum(partials[:, LANES:])
    return total / jnp.maximum(count, 1.0)
